# baseline (device time: 31825 ns/iter reference)
import jax
import jax.numpy as jnp
from jax import lax
from jax.experimental import pallas as pl
from jax.experimental.pallas import tpu as pltpu

N_DEV = 4
N_LAYERS = 3
N_STREAMS = 2
SEND_ORDER = (2, 1, 3)


def _pmod(a):
    return lax.rem(a + N_DEV, N_DEV)


def kernel(x, Win0, Wout0, Win1, Wout1, Win2, Wout2):
    B, D = x.shape
    H = Win0.shape[1]
    bs = B // N_STREAMS
    r8 = bs // N_DEV

    def body(x_ref, win0_ref, wout0_ref, win1_ref, wout1_ref, win2_ref,
             wout2_ref, out_ref, pbuf_ref, xbuf_ref, rsbuf_ref,
             winv_ref, woutv_ref, w_sems, rs_send, rs_recv, ag_send, ag_recv):
        my = lax.axis_index("i")

        Hh = H // 2
        w_specs = [
            (win0_ref, winv_ref.at[0]),
            (wout0_ref, woutv_ref.at[0]),
            (win1_ref, winv_ref.at[1]),
            (wout1_ref, woutv_ref.at[1]),
            (win2_ref, winv_ref.at[2]),
            (wout2_ref.at[pl.ds(0, Hh), :], woutv_ref.at[2, pl.ds(0, Hh), :]),
            (wout2_ref.at[pl.ds(Hh, Hh), :], woutv_ref.at[2, pl.ds(Hh, Hh), :]),
        ]
        w_copies = {}
        w_done = set()

        def w_issue(i):
            if i not in w_copies:
                c = pltpu.make_async_copy(w_specs[i][0], w_specs[i][1],
                                          w_sems.at[i])
                c.start()
                w_copies[i] = c

        def w_wait(i):
            if i not in w_done:
                w_copies[i].wait()
                w_done.add(i)

        w_issue(0)
        w_issue(1)

        def dot(a, b):
            return jnp.dot(a, b, preferred_element_type=jnp.float32)

        def layer(xv, k):
            if k == 0:
                w_wait(0)
                w_issue(2)
                h = jnp.maximum(dot(xv, winv_ref[0]), 0.0)
                w_wait(1)
                w_issue(3)
                return dot(h, woutv_ref[0])
            if k == 1:
                w_wait(2)
                w_issue(4)
                h = jnp.maximum(dot(xv, winv_ref[1]), 0.0)
                w_wait(3)
                w_issue(5)
                w_issue(6)
                return dot(h, woutv_ref[1])
            w_wait(4)
            h = jnp.maximum(dot(xv, winv_ref[2]), 0.0)
            w_wait(5)
            p = dot(h[:, 0:Hh], woutv_ref[2, 0:Hh, :])
            w_wait(6)
            return p + dot(h[:, Hh:H], woutv_ref[2, Hh:H, :])

        drain = []

        def fire_rs(l, s):
            rs = []
            for d in SEND_ORDER:
                t = _pmod(my + d)
                r = pltpu.make_async_remote_copy(
                    src_ref=pbuf_ref.at[l, s, pl.ds(t * r8, r8), :],
                    dst_ref=rsbuf_ref.at[l, s, d - 1],
                    send_sem=rs_send.at[l, s, d - 1],
                    recv_sem=rs_recv.at[l, s, d - 1],
                    device_id=(t,),
                    device_id_type=pl.DeviceIdType.MESH,
                )
                r.start()
                rs.append(r)
            drain.extend(rs)
            return rs

        def reduce_own(l, s, rs):
            for r in rs:
                r.wait_recv()
            return (
                pbuf_ref[l, s, pl.ds(my * r8, r8), :]
                + rsbuf_ref[l, s, 0] + rsbuf_ref[l, s, 1] + rsbuf_ref[l, s, 2]
            )

        def fire_ag(l, s, own):
            xbuf_ref[l, s, pl.ds(my * r8, r8), :] = own
            ag = []
            for d in SEND_ORDER:
                t = _pmod(my + d)
                r = pltpu.make_async_remote_copy(
                    src_ref=xbuf_ref.at[l, s, pl.ds(my * r8, r8), :],
                    dst_ref=xbuf_ref.at[l, s, pl.ds(my * r8, r8), :],
                    send_sem=ag_send.at[l, s, d - 1],
                    recv_sem=ag_recv.at[l, s, d - 1],
                    device_id=(t,),
                    device_id_type=pl.DeviceIdType.MESH,
                )
                r.start()
                ag.append(r)
            drain.extend(ag)
            return ag

        def wait_all(rdmas):
            for r in rdmas:
                r.wait_recv()

        A, Bs = 0, 1

        xa = jnp.concatenate(
            [x_ref[pl.ds(16 * t, r8), :] for t in range(N_DEV)], axis=0
        )
        pbuf_ref[0, A, :, :] = layer(xa, 0)

        barrier_sem = pltpu.get_barrier_semaphore()
        for d in SEND_ORDER:
            pl.semaphore_signal(
                barrier_sem, inc=1,
                device_id=(_pmod(my + d),),
                device_id_type=pl.DeviceIdType.MESH,
            )
        pl.semaphore_wait(barrier_sem, N_DEV - 1)

        rs_a = fire_rs(0, A)
        xb = jnp.concatenate(
            [x_ref[pl.ds(16 * t + r8, r8), :] for t in range(N_DEV)], axis=0
        )
        pbuf_ref[0, Bs, :, :] = layer(xb, 0)
        rs_b = fire_rs(0, Bs)
        ag_a = fire_ag(0, A, reduce_own(0, A, rs_a))
        ag_b = fire_ag(0, Bs, reduce_own(0, Bs, rs_b))

        wait_all(ag_a)
        pbuf_ref[1, A, :, :] = layer(xbuf_ref[0, A, :, :], 1)
        rs_a = fire_rs(1, A)
        wait_all(ag_b)
        pbuf_ref[1, Bs, :, :] = layer(xbuf_ref[0, Bs, :, :], 1)
        rs_b = fire_rs(1, Bs)
        ag_a = fire_ag(1, A, reduce_own(1, A, rs_a))
        ag_b = fire_ag(1, Bs, reduce_own(1, Bs, rs_b))

        wait_all(ag_a)
        pbuf_ref[2, A, :, :] = layer(xbuf_ref[1, A, :, :], 2)
        rs_a = fire_rs(2, A)
        wait_all(ag_b)
        pbuf_ref[2, Bs, :, :] = layer(xbuf_ref[1, Bs, :, :], 2)
        rs_b = fire_rs(2, Bs)
        out_ref[pl.ds(0, r8), :] = reduce_own(2, A, rs_a)
        out_ref[pl.ds(r8, r8), :] = reduce_own(2, Bs, rs_b)

        for r in drain:
            r.wait_send()

    return pl.pallas_call(
        body,
        out_shape=jax.ShapeDtypeStruct((B // N_DEV, D), jnp.float32),
        in_specs=(
            [pl.BlockSpec(memory_space=pltpu.VMEM)]
            + [pl.BlockSpec(memory_space=pl.ANY)] * 6
        ),
        out_specs=pl.BlockSpec(memory_space=pltpu.VMEM),
        scratch_shapes=[
            pltpu.VMEM((N_LAYERS, N_STREAMS, bs, D), jnp.float32),
            pltpu.VMEM((2, N_STREAMS, bs, D), jnp.float32),
            pltpu.VMEM((N_LAYERS, N_STREAMS, N_DEV - 1, r8, D), jnp.float32),
            pltpu.VMEM((N_LAYERS, D, H), jnp.float32),
            pltpu.VMEM((N_LAYERS, H, D), jnp.float32),
            pltpu.SemaphoreType.DMA((7,)),
            pltpu.SemaphoreType.DMA((N_LAYERS, N_STREAMS, N_DEV - 1)),
            pltpu.SemaphoreType.DMA((N_LAYERS, N_STREAMS, N_DEV - 1)),
            pltpu.SemaphoreType.DMA((2, N_STREAMS, N_DEV - 1)),
            pltpu.SemaphoreType.DMA((2, N_STREAMS, N_DEV - 1)),
        ],
        compiler_params=pltpu.CompilerParams(
            vmem_limit_bytes=100 * 1024 * 1024,
            collective_id=0,
        ),
    )(x, Win0, Wout0, Win1, Wout1, Win2, Wout2)


# device time: 31034 ns/iter; 1.0255x vs baseline; 1.0255x over previous
import jax
import jax.numpy as jnp
from jax import lax
from jax.experimental import pallas as pl
from jax.experimental.pallas import tpu as pltpu

N_DEV = 4
N_LAYERS = 3
N_STREAMS = 2
SEND_ORDER = (2, 1, 3)


def _pmod(a):
    return lax.rem(a + N_DEV, N_DEV)


def kernel(x, Win0, Wout0, Win1, Wout1, Win2, Wout2):
    B, D = x.shape
    H = Win0.shape[1]
    bs = B // N_STREAMS
    r8 = bs // N_DEV

    def body(x_ref, win0_ref, wout0_ref, win1_ref, wout1_ref, win2_ref,
             wout2_ref, out_ref, pbuf_ref, xbuf_ref, rsbuf_ref,
             winv_ref, woutv_ref, w_sems, rs_send, rs_recv, ag_send, ag_recv):
        my = lax.axis_index("i")

        Hh = H // 2
        Dh = D // 2
        w_specs = [
            (win0_ref.at[pl.ds(0, Dh), :], winv_ref.at[0, pl.ds(0, Dh), :]),
            (win0_ref.at[pl.ds(Dh, Dh), :], winv_ref.at[0, pl.ds(Dh, Dh), :]),
            (wout0_ref, woutv_ref.at[0]),
            (win1_ref, winv_ref.at[1]),
            (wout1_ref, woutv_ref.at[1]),
            (win2_ref, winv_ref.at[2]),
            (wout2_ref.at[pl.ds(0, Hh), :], woutv_ref.at[2, pl.ds(0, Hh), :]),
            (wout2_ref.at[pl.ds(Hh, Hh), :], woutv_ref.at[2, pl.ds(Hh, Hh), :]),
        ]
        w_copies = {}
        w_done = set()

        def w_issue(i):
            if i not in w_copies:
                c = pltpu.make_async_copy(w_specs[i][0], w_specs[i][1],
                                          w_sems.at[i])
                c.start()
                w_copies[i] = c

        def w_wait(i):
            if i not in w_done:
                w_copies[i].wait()
                w_done.add(i)

        w_issue(0)
        w_issue(1)
        w_issue(2)

        def dot(a, b):
            return jnp.dot(a, b, preferred_element_type=jnp.float32)

        def layer(xv, k):
            if k == 1:
                w_wait(3)
                w_issue(5)
                h = jnp.maximum(dot(xv, winv_ref[1]), 0.0)
                w_wait(4)
                w_issue(6)
                w_issue(7)
                return dot(h, woutv_ref[1])
            w_wait(5)
            h = jnp.maximum(dot(xv, winv_ref[2]), 0.0)
            w_wait(6)
            p = dot(h[:, 0:Hh], woutv_ref[2, 0:Hh, :])
            w_wait(7)
            return p + dot(h[:, Hh:H], woutv_ref[2, Hh:H, :])

        drain = []

        def fire_rs(l, s):
            rs = []
            for d in SEND_ORDER:
                t = _pmod(my + d)
                r = pltpu.make_async_remote_copy(
                    src_ref=pbuf_ref.at[l, s, pl.ds(t * r8, r8), :],
                    dst_ref=rsbuf_ref.at[l, s, d - 1],
                    send_sem=rs_send.at[l, s, d - 1],
                    recv_sem=rs_recv.at[l, s, d - 1],
                    device_id=(t,),
                    device_id_type=pl.DeviceIdType.MESH,
                )
                r.start()
                rs.append(r)
            drain.extend(rs)
            return rs

        def reduce_own(l, s, rs):
            for r in rs:
                r.wait_recv()
            return (
                pbuf_ref[l, s, pl.ds(my * r8, r8), :]
                + rsbuf_ref[l, s, 0] + rsbuf_ref[l, s, 1] + rsbuf_ref[l, s, 2]
            )

        def fire_ag(l, s, own):
            xbuf_ref[l, s, pl.ds(my * r8, r8), :] = own
            ag = []
            for d in SEND_ORDER:
                t = _pmod(my + d)
                r = pltpu.make_async_remote_copy(
                    src_ref=xbuf_ref.at[l, s, pl.ds(my * r8, r8), :],
                    dst_ref=xbuf_ref.at[l, s, pl.ds(my * r8, r8), :],
                    send_sem=ag_send.at[l, s, d - 1],
                    recv_sem=ag_recv.at[l, s, d - 1],
                    device_id=(t,),
                    device_id_type=pl.DeviceIdType.MESH,
                )
                r.start()
                ag.append(r)
            drain.extend(ag)
            return ag

        def wait_all(rdmas):
            for r in rdmas:
                r.wait_recv()

        A, Bs = 0, 1

        xa = jnp.concatenate(
            [x_ref[pl.ds(16 * t, r8), :] for t in range(N_DEV)], axis=0
        )
        xb = jnp.concatenate(
            [x_ref[pl.ds(16 * t + r8, r8), :] for t in range(N_DEV)], axis=0
        )
        w_wait(0)
        qa = dot(xa[:, 0:Dh], winv_ref[0, 0:Dh, :])
        w_wait(1)
        w_issue(3)
        h_a = jnp.maximum(qa + dot(xa[:, Dh:D], winv_ref[0, Dh:D, :]), 0.0)
        h_b = jnp.maximum(dot(xb, winv_ref[0]), 0.0)

        barrier_sem = pltpu.get_barrier_semaphore()
        for d in SEND_ORDER:
            pl.semaphore_signal(
                barrier_sem, inc=1,
                device_id=(_pmod(my + d),),
                device_id_type=pl.DeviceIdType.MESH,
            )
        pl.semaphore_wait(barrier_sem, N_DEV - 1)

        w_wait(2)
        w_issue(4)
        pbuf_ref[0, A, :, :] = dot(h_a, woutv_ref[0])
        rs_a = fire_rs(0, A)
        pbuf_ref[0, Bs, :, :] = dot(h_b, woutv_ref[0])
        rs_b = fire_rs(0, Bs)
        ag_a = fire_ag(0, A, reduce_own(0, A, rs_a))
        ag_b = fire_ag(0, Bs, reduce_own(0, Bs, rs_b))

        wait_all(ag_a)
        pbuf_ref[1, A, :, :] = layer(xbuf_ref[0, A, :, :], 1)
        rs_a = fire_rs(1, A)
        wait_all(ag_b)
        pbuf_ref[1, Bs, :, :] = layer(xbuf_ref[0, Bs, :, :], 1)
        rs_b = fire_rs(1, Bs)
        ag_a = fire_ag(1, A, reduce_own(1, A, rs_a))
        ag_b = fire_ag(1, Bs, reduce_own(1, Bs, rs_b))

        wait_all(ag_a)
        pbuf_ref[2, A, :, :] = layer(xbuf_ref[1, A, :, :], 2)
        rs_a = fire_rs(2, A)
        wait_all(ag_b)
        pbuf_ref[2, Bs, :, :] = layer(xbuf_ref[1, Bs, :, :], 2)
        rs_b = fire_rs(2, Bs)
        out_ref[pl.ds(0, r8), :] = reduce_own(2, A, rs_a)
        out_ref[pl.ds(r8, r8), :] = reduce_own(2, Bs, rs_b)

        for r in drain:
            r.wait_send()

    return pl.pallas_call(
        body,
        out_shape=jax.ShapeDtypeStruct((B // N_DEV, D), jnp.float32),
        in_specs=(
            [pl.BlockSpec(memory_space=pltpu.VMEM)]
            + [pl.BlockSpec(memory_space=pl.ANY)] * 6
        ),
        out_specs=pl.BlockSpec(memory_space=pltpu.VMEM),
        scratch_shapes=[
            pltpu.VMEM((N_LAYERS, N_STREAMS, bs, D), jnp.float32),
            pltpu.VMEM((2, N_STREAMS, bs, D), jnp.float32),
            pltpu.VMEM((N_LAYERS, N_STREAMS, N_DEV - 1, r8, D), jnp.float32),
            pltpu.VMEM((N_LAYERS, D, H), jnp.float32),
            pltpu.VMEM((N_LAYERS, H, D), jnp.float32),
            pltpu.SemaphoreType.DMA((8,)),
            pltpu.SemaphoreType.DMA((N_LAYERS, N_STREAMS, N_DEV - 1)),
            pltpu.SemaphoreType.DMA((N_LAYERS, N_STREAMS, N_DEV - 1)),
            pltpu.SemaphoreType.DMA((2, N_STREAMS, N_DEV - 1)),
            pltpu.SemaphoreType.DMA((2, N_STREAMS, N_DEV - 1)),
        ],
        compiler_params=pltpu.CompilerParams(
            vmem_limit_bytes=100 * 1024 * 1024,
            collective_id=0,
        ),
    )(x, Win0, Wout0, Win1, Wout1, Win2, Wout2)
